# baseline (speedup 1.0000x reference)
import functools

import jax
import jax.numpy as jnp
from jax import lax
from jax.experimental import pallas as pl
from jax.experimental.pallas import tpu as pltpu
from jax.experimental.pallas import tpu_sc as plsc

VOCAB = 100000
EMBED = 64
BATCH = 1024
CTX = 50

_NC = 2
_NS = 16
_NW = _NC * _NS
_LANES = 16
_ROWS_PER_W = BATCH // _NW


def _sc_body(tok_hbm, table_hbm, out_hbm, idx_v, rows_v, mean_v, sem):
    wid = lax.axis_index("s") * _NC + lax.axis_index("c")
    base = wid * _ROWS_PER_W

    pltpu.sync_copy(tok_hbm.at[pl.ds(base, _ROWS_PER_W)], idx_v)

    inv_ctx = jnp.float32(1.0 / CTX)

    def _gather(r, buf):
        return pltpu.async_copy(table_hbm.at[idx_v.at[r]], rows_v.at[buf], sem)

    pending = _gather(0, 0)
    for r in range(_ROWS_PER_W):
        buf = r % 2
        if r + 1 < _ROWS_PER_W:
            nxt = _gather(r + 1, 1 - buf)
        pending.wait()
        if r + 1 < _ROWS_PER_W:
            pending = nxt

        def body(c, acc, buf=buf):
            return tuple(
                acc[d] + rows_v[buf, c, pl.ds(d * _LANES, _LANES)]
                for d in range(EMBED // _LANES)
            )

        zeros = tuple(
            jnp.zeros((_LANES,), jnp.float32) for _ in range(EMBED // _LANES)
        )
        acc = lax.fori_loop(0, CTX, body, zeros)
        for d in range(EMBED // _LANES):
            mean_v[r, pl.ds(d * _LANES, _LANES)] = acc[d] * inv_ctx

    pltpu.sync_copy(mean_v, out_hbm.at[pl.ds(base, _ROWS_PER_W)])


def _sc_gather_mean(context_tokens, emb_table):
    mesh = plsc.VectorSubcoreMesh(core_axis_name="c", subcore_axis_name="s")
    k = functools.partial(
        pl.kernel,
        mesh=mesh,
        out_type=jax.ShapeDtypeStruct((BATCH, EMBED), jnp.float32),
        scratch_types=[
            pltpu.VMEM((_ROWS_PER_W, CTX), jnp.int32),
            pltpu.VMEM((2, CTX, EMBED), jnp.float32),
            pltpu.VMEM((_ROWS_PER_W, EMBED), jnp.float32),
            pltpu.SemaphoreType.DMA,
        ],
        compiler_params=pltpu.CompilerParams(use_tc_tiling_on_sc=False),
    )(_sc_body)
    return k(context_tokens, emb_table)


_BV = 2048
_NSLOT = 4
_NFULL = VOCAB // _BV
_TAIL = VOCAB - _NFULL * _BV
_NSTEP = _NFULL + 1


def _mm_body(means_ref, w_ref, b_ref, out_hbm, acc_ref, sems):
    j = pl.program_id(0)
    slot = lax.rem(j, _NSLOT)

    def _copy(step, slot):
        return pltpu.make_async_copy(
            acc_ref.at[slot],
            out_hbm.at[pl.ds(step * _BV, _BV)],
            sems.at[slot],
        )

    def _tail_copy(slot):
        return pltpu.make_async_copy(
            acc_ref.at[slot, pl.ds(0, _TAIL)],
            out_hbm.at[pl.ds(_NFULL * _BV, _TAIL)],
            sems.at[slot],
        )

    @pl.when(j >= _NSLOT)
    def _():
        _copy(j - _NSLOT, slot).wait()

    acc_ref[slot] = (
        lax.dot_general(
            w_ref[...],
            means_ref[...],
            (((1,), (1,)), ((), ())),
            preferred_element_type=jnp.float32,
        )
        + b_ref[...]
    )

    @pl.when(j < _NFULL)
    def _():
        _copy(j, slot).start()

    @pl.when(j == _NFULL)
    def _():
        _tail_copy(_NFULL % _NSLOT).start()
        for back in range(1, _NSLOT):
            step = _NFULL - back
            _copy(step, step % _NSLOT).wait()
        _tail_copy(_NFULL % _NSLOT).wait()


def _tc_project(means, w_score, b_score2d):
    return pl.pallas_call(
        _mm_body,
        grid=(_NSTEP,),
        in_specs=[
            pl.BlockSpec((BATCH, EMBED), lambda j: (0, 0)),
            pl.BlockSpec((_BV, EMBED), lambda j: (j, 0)),
            pl.BlockSpec((_BV, 1), lambda j: (j, 0)),
        ],
        out_specs=pl.BlockSpec(memory_space=pl.ANY),
        out_shape=jax.ShapeDtypeStruct((VOCAB, BATCH), jnp.float32),
        scratch_shapes=[
            pltpu.VMEM((_NSLOT, _BV, BATCH), jnp.float32),
            pltpu.SemaphoreType.DMA((_NSLOT,)),
        ],
        compiler_params=pltpu.CompilerParams(
            dimension_semantics=("arbitrary",),
        ),
    )(means, w_score, b_score2d)


def kernel(context_tokens, emb_table, w_score, b_score):
    means = _sc_gather_mean(context_tokens.astype(jnp.int32), emb_table)
    yt = _tc_project(means, w_score, b_score.reshape(VOCAB, 1))
    return yt.T


# speedup vs baseline: 1.0088x; 1.0088x over previous
import functools

import jax
import jax.numpy as jnp
from jax import lax
from jax.experimental import pallas as pl
from jax.experimental.pallas import tpu as pltpu
from jax.experimental.pallas import tpu_sc as plsc

VOCAB = 100000
EMBED = 64
BATCH = 1024
CTX = 50

_NC = 2
_NS = 16
_NW = _NC * _NS
_LANES = 16
_ROWS_PER_W = BATCH // _NW


def _sc_body(tok_hbm, table_hbm, out_hbm, idx_v, rows_v, mean_v, sem):
    wid = lax.axis_index("s") * _NC + lax.axis_index("c")
    base = wid * _ROWS_PER_W

    pltpu.sync_copy(tok_hbm.at[pl.ds(base, _ROWS_PER_W)], idx_v)

    inv_ctx = jnp.float32(1.0 / CTX)

    def _gather(r, buf):
        return pltpu.async_copy(table_hbm.at[idx_v.at[r]], rows_v.at[buf], sem)

    pending = _gather(0, 0)
    for r in range(_ROWS_PER_W):
        buf = r % 2
        if r + 1 < _ROWS_PER_W:
            nxt = _gather(r + 1, 1 - buf)
        pending.wait()
        if r + 1 < _ROWS_PER_W:
            pending = nxt

        def body(c5, acc, buf=buf):
            for k in range(5):
                c = c5 * 5 + k
                acc = tuple(
                    acc[d] + rows_v[buf, c, pl.ds(d * _LANES, _LANES)]
                    for d in range(EMBED // _LANES)
                )
            return acc

        zeros = tuple(
            jnp.zeros((_LANES,), jnp.float32) for _ in range(EMBED // _LANES)
        )
        acc = lax.fori_loop(0, CTX // 5, body, zeros)
        for d in range(EMBED // _LANES):
            mean_v[r, pl.ds(d * _LANES, _LANES)] = acc[d] * inv_ctx

    pltpu.sync_copy(mean_v, out_hbm.at[pl.ds(base, _ROWS_PER_W)])


def _sc_gather_mean(context_tokens, emb_table):
    mesh = plsc.VectorSubcoreMesh(core_axis_name="c", subcore_axis_name="s")
    k = functools.partial(
        pl.kernel,
        mesh=mesh,
        out_type=jax.ShapeDtypeStruct((BATCH, EMBED), jnp.float32),
        scratch_types=[
            pltpu.VMEM((_ROWS_PER_W, CTX), jnp.int32),
            pltpu.VMEM((2, CTX, EMBED), jnp.float32),
            pltpu.VMEM((_ROWS_PER_W, EMBED), jnp.float32),
            pltpu.SemaphoreType.DMA,
        ],
        compiler_params=pltpu.CompilerParams(use_tc_tiling_on_sc=False),
    )(_sc_body)
    return k(context_tokens, emb_table)


_BV = 4096
_NSLOT = 3
_NFULL = VOCAB // _BV
_TAIL = VOCAB - _NFULL * _BV
_NSTEP = _NFULL + 1


def _mm_body(means_ref, w_ref, b_ref, out_hbm, acc_ref, sems):
    j = pl.program_id(0)
    slot = lax.rem(j, _NSLOT)

    def _copy(step, slot):
        return pltpu.make_async_copy(
            acc_ref.at[slot],
            out_hbm.at[pl.ds(step * _BV, _BV)],
            sems.at[slot],
        )

    def _tail_copy(slot):
        return pltpu.make_async_copy(
            acc_ref.at[slot, pl.ds(0, _TAIL)],
            out_hbm.at[pl.ds(_NFULL * _BV, _TAIL)],
            sems.at[slot],
        )

    @pl.when(j >= _NSLOT)
    def _():
        _copy(j - _NSLOT, slot).wait()

    acc_ref[slot] = (
        lax.dot_general(
            w_ref[...],
            means_ref[...],
            (((1,), (1,)), ((), ())),
            preferred_element_type=jnp.float32,
        )
        + b_ref[...]
    )

    @pl.when(j < _NFULL)
    def _():
        _copy(j, slot).start()

    @pl.when(j == _NFULL)
    def _():
        _tail_copy(_NFULL % _NSLOT).start()
        for back in range(1, _NSLOT):
            step = _NFULL - back
            _copy(step, step % _NSLOT).wait()
        _tail_copy(_NFULL % _NSLOT).wait()


def _tc_project(means, w_score, b_score2d):
    return pl.pallas_call(
        _mm_body,
        grid=(_NSTEP,),
        in_specs=[
            pl.BlockSpec((BATCH, EMBED), lambda j: (0, 0)),
            pl.BlockSpec((_BV, EMBED), lambda j: (j, 0)),
            pl.BlockSpec((_BV, 1), lambda j: (j, 0)),
        ],
        out_specs=pl.BlockSpec(memory_space=pl.ANY),
        out_shape=jax.ShapeDtypeStruct((VOCAB, BATCH), jnp.float32),
        scratch_shapes=[
            pltpu.VMEM((_NSLOT, _BV, BATCH), jnp.float32),
            pltpu.SemaphoreType.DMA((_NSLOT,)),
        ],
        compiler_params=pltpu.CompilerParams(
            dimension_semantics=("arbitrary",),
            vmem_limit_bytes=60 * 1024 * 1024,
        ),
    )(means, w_score, b_score2d)


def kernel(context_tokens, emb_table, w_score, b_score):
    means = _sc_gather_mean(context_tokens.astype(jnp.int32), emb_table)
    yt = _tc_project(means, w_score, b_score.reshape(VOCAB, 1))
    return yt.T
